# Initial kernel scaffold; baseline (speedup 1.0000x reference)
import jax
import jax.numpy as jnp
from jax.experimental import pallas as pl


def kernel(x, edge_index, W, b):
    raise NotImplementedError("write your pallas kernel here")


# speedup vs baseline: 4.5572x; 4.5572x over previous
import functools

import jax
import jax.numpy as jnp
from jax import lax
from jax.experimental import pallas as pl
from jax.experimental.pallas import tpu as pltpu
from jax.experimental.pallas import tpu_sc as plsc

N = 10000
L = 8
D = 128
OUT = 300
CHUNK = 128
NCHUNK = 3
OUT_PAD = CHUNK * NCHUNK
E = 320000

B = 128
NB = E // B
NSUB = 16
NB_BASE = NB // NSUB
NB_REM = NB % NSUB
NPAD = 10240
ROW_CHUNK = 128
NCHUNK_FULL = N // ROW_CHUNK
TAIL_BASE = NCHUNK_FULL * ROW_CHUNK
TAIL_ROWS = N - TAIL_BASE
VECS = CHUNK // 16


def _tc_linear(x, w_pad, b_pad):
    blk = 400

    def body(x_ref, w_ref, b_ref, h0_ref, h1_ref, h2_ref):
        xm = jnp.mean(x_ref[...], axis=1)
        h = jnp.dot(xm, w_ref[...], preferred_element_type=jnp.float32)
        h = h + b_ref[...]
        h0_ref[...] = h[:, :CHUNK]
        h1_ref[...] = h[:, CHUNK:2 * CHUNK]
        h2_ref[...] = h[:, 2 * CHUNK:]

    return pl.pallas_call(
        body,
        grid=(N // blk,),
        in_specs=[
            pl.BlockSpec((blk, L, D), lambda i: (i, 0, 0)),
            pl.BlockSpec((D, OUT_PAD), lambda i: (0, 0)),
            pl.BlockSpec((1, OUT_PAD), lambda i: (0, 0)),
        ],
        out_specs=[
            pl.BlockSpec((blk, CHUNK), lambda i: (i, 0)),
            pl.BlockSpec((blk, CHUNK), lambda i: (i, 0)),
            pl.BlockSpec((blk, CHUNK), lambda i: (i, 0)),
        ],
        out_shape=[
            jax.ShapeDtypeStruct((N, CHUNK), jnp.float32),
            jax.ShapeDtypeStruct((N, CHUNK), jnp.float32),
            jax.ShapeDtypeStruct((N, CHUNK), jnp.float32),
        ],
    )(x, w_pad, b_pad)


def _rsqrt16(v):
    i = lax.bitcast_convert_type(v, jnp.int32)
    i = jnp.int32(0x5F3759DF) - (i >> 1)
    y = lax.bitcast_convert_type(i, jnp.float32)
    for _ in range(3):
        y = y * (1.5 - 0.5 * v * y * y)
    return y


def _sc_gcn(row, col, h0, h1, h2):
    mesh = plsc.VectorSubcoreMesh(core_axis_name="c", subcore_axis_name="s")

    @functools.partial(
        pl.kernel,
        out_type=jax.ShapeDtypeStruct((NCHUNK, N, CHUNK), jnp.float32),
        mesh=mesh,
        compiler_params=pltpu.CompilerParams(
            needs_layout_passes=False, use_tc_tiling_on_sc=False),
        scratch_types=dict(
            deg_sh=pltpu.VMEM_SHARED((NPAD,), jnp.float32),
            out_sh=pltpu.VMEM_SHARED((N, CHUNK), jnp.float32),
            dis=pltpu.VMEM((NPAD,), jnp.float32),
            rowb=pltpu.VMEM((2, B), jnp.int32),
            colb=pltpu.VMEM((2, B), jnp.int32),
            gbuf=pltpu.VMEM((2, B, CHUNK), jnp.float32),
            normb=pltpu.VMEM((B + 16,), jnp.float32),
        ),
    )
    def k(row_hbm, col_hbm, h0_hbm, h1_hbm, h2_hbm, out_hbm,
          deg_sh, out_sh, dis, rowb, colb, gbuf, normb):
        c = lax.axis_index("c")
        s = lax.axis_index("s")
        nb_s = NB_BASE + jnp.where(s < NB_REM, 1, 0)
        start_s = NB_BASE * s + jnp.minimum(s, NB_REM)

        @pl.loop(0, NPAD // 16)
        def _(j):
            dis[pl.ds(16 * j, 16)] = jnp.zeros((16,), jnp.float32)

        @pl.when(s == 0)
        def _():
            pltpu.sync_copy(dis, deg_sh)

        plsc.subcore_barrier()

        @pl.loop(0, nb_s)
        def _(jb):
            e0 = (start_s + jb) * B
            pltpu.sync_copy(row_hbm.at[pl.ds(e0, B)], rowb.at[0])
            pltpu.sync_copy(col_hbm.at[pl.ds(e0, B)], colb.at[0])
            for v in range(B // 16):
                sl = pl.ds(16 * v, 16)
                rv = rowb[0, sl]
                cv = colb[0, sl]
                normb[sl] = jnp.where(rv != cv, 1.0, 0.0)
            pltpu.sync_copy(normb.at[pl.ds(0, B)], deg_sh.at[rowb.at[0]],
                            add=True)

        plsc.subcore_barrier()

        pltpu.sync_copy(deg_sh, dis)

        @pl.loop(0, NPAD // 16)
        def _(j):
            sl = pl.ds(16 * j, 16)
            dis[sl] = _rsqrt16(dis[sl] + 1.0)

        nch_s = (NCHUNK_FULL // NSUB) + jnp.where(s < NCHUNK_FULL % NSUB, 1, 0)

        def chunk_pass(h_hbm, out_idx):
            def self_term_chunk(base, nrows):
                pltpu.sync_copy(h_hbm.at[pl.ds(base, nrows)],
                                gbuf.at[0].at[pl.ds(0, nrows)])

                @pl.loop(0, nrows)
                def _(r):
                    d = dis[pl.ds(base + r, 16)][0]
                    d2 = d * d
                    for v in range(VECS):
                        sl = pl.ds(16 * v, 16)
                        gbuf[0, r, sl] = gbuf[0, r, sl] * d2

                pltpu.sync_copy(gbuf.at[0].at[pl.ds(0, nrows)],
                                out_sh.at[pl.ds(base, nrows)])

            @pl.loop(0, nch_s)
            def _(t):
                base = pl.multiple_of(ROW_CHUNK * (s + NSUB * t), ROW_CHUNK)
                self_term_chunk(base, ROW_CHUNK)

            @pl.when(s == NSUB - 1)
            def _():
                self_term_chunk(TAIL_BASE, TAIL_ROWS)

            plsc.subcore_barrier()

            @pl.loop(0, nb_s)
            def _(jb):
                e0 = (start_s + jb) * B
                pltpu.sync_copy(row_hbm.at[pl.ds(e0, B)], rowb.at[0])
                pltpu.sync_copy(col_hbm.at[pl.ds(e0, B)], colb.at[0])
                pltpu.sync_copy(h_hbm.at[rowb.at[0]], gbuf.at[0])
                for v in range(B // 16):
                    sl = pl.ds(16 * v, 16)
                    rv = rowb[0, sl]
                    cv = colb[0, sl]
                    nr = plsc.load_gather(dis, [rv])
                    nc = plsc.load_gather(dis, [cv])
                    normb[sl] = jnp.where(rv != cv, nr * nc, 0.0)

                @pl.loop(0, B)
                def _(e):
                    ne = normb[pl.ds(e, 16)][0]
                    for v in range(VECS):
                        sl = pl.ds(16 * v, 16)
                        gbuf[0, e, sl] = gbuf[0, e, sl] * ne

                pltpu.sync_copy(gbuf.at[0], out_sh.at[colb.at[0]], add=True)

            plsc.subcore_barrier()

            def writeout_chunk(base, nrows):
                pltpu.sync_copy(out_sh.at[pl.ds(base, nrows)],
                                gbuf.at[0].at[pl.ds(0, nrows)])
                pltpu.sync_copy(gbuf.at[0].at[pl.ds(0, nrows)],
                                out_hbm.at[out_idx].at[pl.ds(base, nrows)])

            @pl.loop(0, nch_s)
            def _(t):
                base = pl.multiple_of(ROW_CHUNK * (s + NSUB * t), ROW_CHUNK)
                writeout_chunk(base, ROW_CHUNK)

            @pl.when(s == NSUB - 1)
            def _():
                writeout_chunk(TAIL_BASE, TAIL_ROWS)

        @pl.when(c == 0)
        def _():
            chunk_pass(h0_hbm, 0)
            plsc.subcore_barrier()
            chunk_pass(h1_hbm, 1)

        @pl.when(c == 1)
        def _():
            chunk_pass(h2_hbm, 2)

    return k(row, col, h0, h1, h2)


def kernel(x, edge_index, W, b):
    w_pad = jnp.pad(W, ((0, 0), (0, OUT_PAD - OUT)))
    b_pad = jnp.pad(b, (0, OUT_PAD - OUT)).reshape(1, OUT_PAD)
    h0, h1, h2 = _tc_linear(x, w_pad, b_pad)
    row = edge_index[0].astype(jnp.int32)
    col = edge_index[1].astype(jnp.int32)
    out3 = _sc_gcn(row, col, h0, h1, h2)
    return jnp.concatenate([out3[0], out3[1], out3[2]], axis=1)[:, :OUT]


# speedup vs baseline: 6.0084x; 1.3185x over previous
import functools

import jax
import jax.numpy as jnp
from jax import lax
from jax.experimental import pallas as pl
from jax.experimental.pallas import tpu as pltpu
from jax.experimental.pallas import tpu_sc as plsc

N = 10000
L = 8
D = 128
OUT = 300
CHUNK = 128
NCHUNK = 3
OUT_PAD = CHUNK * NCHUNK
E = 320000

B = 128
NB = E // B
NSUB = 16
NB_BASE = NB // NSUB
NB_REM = NB % NSUB
NPAD = 10240
NACC = N + 16
ROW_CHUNK = 128
NCHUNK_FULL = N // ROW_CHUNK
TAIL_BASE = NCHUNK_FULL * ROW_CHUNK
TAIL_ROWS = N - TAIL_BASE
VECS = CHUNK // 16


def _tc_linear(x, w_pad, b_pad):
    blk = 400

    def body(x_ref, w_ref, b_ref, h0_ref, h1_ref, h2_ref):
        xm = jnp.mean(x_ref[...], axis=1)
        h = jnp.dot(xm, w_ref[...], preferred_element_type=jnp.float32)
        h = h + b_ref[...]
        h0_ref[...] = h[:, :CHUNK]
        h1_ref[...] = h[:, CHUNK:2 * CHUNK]
        h2_ref[...] = h[:, 2 * CHUNK:]

    return pl.pallas_call(
        body,
        grid=(N // blk,),
        in_specs=[
            pl.BlockSpec((blk, L, D), lambda i: (i, 0, 0)),
            pl.BlockSpec((D, OUT_PAD), lambda i: (0, 0)),
            pl.BlockSpec((1, OUT_PAD), lambda i: (0, 0)),
        ],
        out_specs=[
            pl.BlockSpec((blk, CHUNK), lambda i: (i, 0)),
            pl.BlockSpec((blk, CHUNK), lambda i: (i, 0)),
            pl.BlockSpec((blk, CHUNK), lambda i: (i, 0)),
        ],
        out_shape=[
            jax.ShapeDtypeStruct((N, CHUNK), jnp.float32),
            jax.ShapeDtypeStruct((N, CHUNK), jnp.float32),
            jax.ShapeDtypeStruct((N, CHUNK), jnp.float32),
        ],
    )(x, w_pad, b_pad)


def _rsqrt16(v):
    i = lax.bitcast_convert_type(v, jnp.int32)
    i = jnp.int32(0x5F3759DF) - (i >> 1)
    y = lax.bitcast_convert_type(i, jnp.float32)
    for _ in range(3):
        y = y * (1.5 - 0.5 * v * y * y)
    return y


def _sc_gcn(row, col, h0, h1, h2):
    mesh = plsc.VectorSubcoreMesh(core_axis_name="c", subcore_axis_name="s")

    @functools.partial(
        pl.kernel,
        out_type=(
            jax.ShapeDtypeStruct((NCHUNK, N, CHUNK), jnp.float32),
            jax.ShapeDtypeStruct((N, CHUNK), jnp.float32),
            jax.ShapeDtypeStruct((N, CHUNK), jnp.float32),
            jax.ShapeDtypeStruct((N, CHUNK), jnp.float32),
        ),
        mesh=mesh,
        compiler_params=pltpu.CompilerParams(
            needs_layout_passes=False, use_tc_tiling_on_sc=False),
        scratch_types=dict(
            deg_sh=pltpu.VMEM_SHARED((NPAD,), jnp.float32),
            out_sh=pltpu.VMEM_SHARED((NACC, CHUNK), jnp.float32),
            dis=pltpu.VMEM((NPAD,), jnp.float32),
            rowb=pltpu.VMEM((2, B), jnp.int32),
            colb=pltpu.VMEM((2, B), jnp.int32),
            gbuf=pltpu.VMEM((2, B, CHUNK), jnp.float32),
            oneb=pltpu.VMEM((B,), jnp.float32),
        ),
    )
    def k(row_hbm, col_hbm, h0_hbm, h1_hbm, h2_hbm,
          out_hbm, hp0_hbm, hp1_hbm, hp2_hbm,
          deg_sh, out_sh, dis, rowb, colb, gbuf, oneb):
        c = lax.axis_index("c")
        s = lax.axis_index("s")
        nb_s = NB_BASE + jnp.where(s < NB_REM, 1, 0)
        start_s = NB_BASE * s + jnp.minimum(s, NB_REM)

        @pl.loop(0, NPAD // 16)
        def _(j):
            dis[pl.ds(16 * j, 16)] = jnp.zeros((16,), jnp.float32)

        @pl.when(s == 0)
        def _():
            pltpu.sync_copy(dis, deg_sh)

        plsc.subcore_barrier()

        @pl.loop(0, nb_s)
        def _(jb):
            e0 = (start_s + jb) * B
            pltpu.sync_copy(row_hbm.at[pl.ds(e0, B)], rowb.at[0])
            pltpu.sync_copy(col_hbm.at[pl.ds(e0, B)], colb.at[0])
            for v in range(B // 16):
                sl = pl.ds(16 * v, 16)
                rv = rowb[0, sl]
                cv = colb[0, sl]
                oneb[sl] = jnp.where(rv != cv, 1.0, 0.0)
            pltpu.sync_copy(oneb, deg_sh.at[rowb.at[0]], add=True)

        plsc.subcore_barrier()

        pltpu.sync_copy(deg_sh, dis)

        @pl.loop(0, NPAD // 16)
        def _(j):
            sl = pl.ds(16 * j, 16)
            dis[sl] = _rsqrt16(dis[sl] + 1.0)

        nch_s = (NCHUNK_FULL // NSUB) + jnp.where(s < NCHUNK_FULL % NSUB, 1, 0)

        def chunk_pass(h_hbm, hp_hbm, out_idx):
            def scale_chunk(base, nrows):
                pltpu.sync_copy(h_hbm.at[pl.ds(base, nrows)],
                                gbuf.at[0].at[pl.ds(0, nrows)])

                @pl.loop(0, nrows)
                def _(r):
                    d = dis[pl.ds(base + r, 16)][0]
                    for v in range(VECS):
                        sl = pl.ds(16 * v, 16)
                        gbuf[0, r, sl] = gbuf[0, r, sl] * d

                pltpu.sync_copy(gbuf.at[0].at[pl.ds(0, nrows)],
                                hp_hbm.at[pl.ds(base, nrows)])
                pltpu.sync_copy(gbuf.at[0].at[pl.ds(0, nrows)],
                                out_sh.at[pl.ds(base, nrows)])

            @pl.loop(0, nch_s)
            def _(t):
                base = pl.multiple_of(ROW_CHUNK * (s + NSUB * t), ROW_CHUNK)
                scale_chunk(base, ROW_CHUNK)

            @pl.when(s == NSUB - 1)
            def _():
                scale_chunk(TAIL_BASE, TAIL_ROWS)

            plsc.subcore_barrier()

            @pl.loop(0, nb_s)
            def _(jb):
                e0 = (start_s + jb) * B
                pltpu.sync_copy(row_hbm.at[pl.ds(e0, B)], rowb.at[0])
                pltpu.sync_copy(col_hbm.at[pl.ds(e0, B)], colb.at[0])
                for v in range(B // 16):
                    sl = pl.ds(16 * v, 16)
                    rv = rowb[0, sl]
                    cv = colb[0, sl]
                    colb[0, sl] = jnp.where(rv != cv, cv, N)
                pltpu.sync_copy(hp_hbm.at[rowb.at[0]], gbuf.at[0])
                pltpu.sync_copy(gbuf.at[0], out_sh.at[colb.at[0]], add=True)

            plsc.subcore_barrier()

            def writeout_chunk(base, nrows):
                pltpu.sync_copy(out_sh.at[pl.ds(base, nrows)],
                                gbuf.at[0].at[pl.ds(0, nrows)])

                @pl.loop(0, nrows)
                def _(r):
                    d = dis[pl.ds(base + r, 16)][0]
                    for v in range(VECS):
                        sl = pl.ds(16 * v, 16)
                        gbuf[0, r, sl] = gbuf[0, r, sl] * d

                pltpu.sync_copy(gbuf.at[0].at[pl.ds(0, nrows)],
                                out_hbm.at[out_idx].at[pl.ds(base, nrows)])

            @pl.loop(0, nch_s)
            def _(t):
                base = pl.multiple_of(ROW_CHUNK * (s + NSUB * t), ROW_CHUNK)
                writeout_chunk(base, ROW_CHUNK)

            @pl.when(s == NSUB - 1)
            def _():
                writeout_chunk(TAIL_BASE, TAIL_ROWS)

        @pl.when(c == 0)
        def _():
            chunk_pass(h0_hbm, hp0_hbm, 0)
            plsc.subcore_barrier()
            chunk_pass(h1_hbm, hp1_hbm, 1)

        @pl.when(c == 1)
        def _():
            chunk_pass(h2_hbm, hp2_hbm, 2)

    return k(row, col, h0, h1, h2)


def kernel(x, edge_index, W, b):
    w_pad = jnp.pad(W, ((0, 0), (0, OUT_PAD - OUT)))
    b_pad = jnp.pad(b, (0, OUT_PAD - OUT)).reshape(1, OUT_PAD)
    h0, h1, h2 = _tc_linear(x, w_pad, b_pad)
    row = edge_index[0].astype(jnp.int32)
    col = edge_index[1].astype(jnp.int32)
    out3, _, _, _ = _sc_gcn(row, col, h0, h1, h2)
    return jnp.concatenate([out3[0], out3[1], out3[2]], axis=1)[:, :OUT]


# speedup vs baseline: 13.1708x; 2.1920x over previous
import functools

import jax
import jax.numpy as jnp
from jax import lax
from jax.experimental import pallas as pl
from jax.experimental.pallas import tpu as pltpu
from jax.experimental.pallas import tpu_sc as plsc

N = 10000
L = 8
D = 128
OUT = 300
CHUNK = 128
NCHUNK = 3
OUT_PAD = CHUNK * NCHUNK
E = 320000

B = 128
NB = E // B
NSUB = 16
NBQ = NB // NSUB
RB = 52
NROUND = NBQ // RB
NB_REM = NB % NSUB
NPAD = 10240
NACC = N + 8
NHP = N + 16
ROW_CHUNK = 128
NCHUNK_FULL = N // ROW_CHUNK
TAIL_BASE = NCHUNK_FULL * ROW_CHUNK
TAIL_ROWS = N - TAIL_BASE
VECS = CHUNK // 16
DSLICE = NPAD // NSUB


def _tc_linear(x, w_pad, b_pad):
    blk = 400

    def body(x_ref, w_ref, b_ref, h0_ref, h1_ref, h2_ref):
        xm = jnp.mean(x_ref[...], axis=1)
        h = jnp.dot(xm, w_ref[...], preferred_element_type=jnp.float32)
        h = h + b_ref[...]
        h0_ref[...] = h[:, :CHUNK]
        h1_ref[...] = h[:, CHUNK:2 * CHUNK]
        h2_ref[...] = h[:, 2 * CHUNK:]

    return pl.pallas_call(
        body,
        grid=(N // blk,),
        in_specs=[
            pl.BlockSpec((blk, L, D), lambda i: (i, 0, 0)),
            pl.BlockSpec((D, OUT_PAD), lambda i: (0, 0)),
            pl.BlockSpec((1, OUT_PAD), lambda i: (0, 0)),
        ],
        out_specs=[
            pl.BlockSpec((blk, CHUNK), lambda i: (i, 0)),
            pl.BlockSpec((blk, CHUNK), lambda i: (i, 0)),
            pl.BlockSpec((blk, CHUNK), lambda i: (i, 0)),
        ],
        out_shape=[
            jax.ShapeDtypeStruct((N, CHUNK), jnp.float32),
            jax.ShapeDtypeStruct((N, CHUNK), jnp.float32),
            jax.ShapeDtypeStruct((N, CHUNK), jnp.float32),
        ],
    )(x, w_pad, b_pad)


def _rsqrt16(v):
    i = lax.bitcast_convert_type(v, jnp.int32)
    i = jnp.int32(0x5F3759DF) - (i >> 1)
    y = lax.bitcast_convert_type(i, jnp.float32)
    for _ in range(3):
        y = y * (1.5 - 0.5 * v * y * y)
    return y


def _sc_gcn(row2d, col2d, h0, h1, h2):
    mesh = plsc.VectorSubcoreMesh(core_axis_name="c", subcore_axis_name="s")

    @functools.partial(
        pl.kernel,
        out_type=(
            jax.ShapeDtypeStruct((NCHUNK, N, CHUNK), jnp.float32),
            jax.ShapeDtypeStruct((NHP, CHUNK), jnp.float32),
            jax.ShapeDtypeStruct((NHP, CHUNK), jnp.float32),
            jax.ShapeDtypeStruct((NHP, CHUNK), jnp.float32),
        ),
        mesh=mesh,
        compiler_params=pltpu.CompilerParams(
            needs_layout_passes=False, use_tc_tiling_on_sc=False),
        scratch_types=dict(
            deg_sh=pltpu.VMEM_SHARED((NPAD,), jnp.float32),
            out_sh=pltpu.VMEM_SHARED((NACC, CHUNK), jnp.float32),
            disb=pltpu.VMEM((DSLICE,), jnp.float32),
            dchunk=pltpu.VMEM((ROW_CHUNK + 16,), jnp.float32),
            rowp=pltpu.VMEM((RB, B), jnp.int32),
            colp=pltpu.VMEM((RB, B), jnp.int32),
            rowb=pltpu.VMEM((2, B), jnp.int32),
            colb=pltpu.VMEM((2, B), jnp.int32),
            oneb=pltpu.VMEM((B,), jnp.float32),
            gbuf=pltpu.VMEM((2, B, CHUNK), jnp.float32),
            gsem=pltpu.SemaphoreType.DMA,
            dsem=pltpu.SemaphoreType.DMA,
        ),
    )
    def k(row_hbm, col_hbm, h0_hbm, h1_hbm, h2_hbm,
          out_hbm, hp0_hbm, hp1_hbm, hp2_hbm,
          deg_sh, out_sh, disb, dchunk, rowp, colp, rowb, colb, oneb, gbuf,
          gsem, dsem):
        c = lax.axis_index("c")
        s = lax.axis_index("s")
        b0 = NBQ * s
        rem_b = NB - NB_REM + s

        def load_round(r):
            pltpu.sync_copy(row_hbm.at[pl.ds(b0 + r * RB, RB)], rowp)
            pltpu.sync_copy(col_hbm.at[pl.ds(b0 + r * RB, RB)], colp)

            @pl.loop(0, RB)
            def _(jb):
                for v in range(B // 16):
                    sl = pl.ds(16 * v, 16)
                    rv = rowp[jb, sl]
                    cv = colp[jb, sl]
                    nonself = rv != cv
                    rowp[jb, sl] = jnp.where(nonself, rv, N)
                    colp[jb, sl] = jnp.where(nonself, cv, N)

        @pl.loop(0, DSLICE // 16)
        def _(j):
            disb[pl.ds(16 * j, 16)] = jnp.zeros((16,), jnp.float32)

        pltpu.sync_copy(disb, deg_sh.at[pl.ds(s * DSLICE, DSLICE)])
        for v in range(B // 16):
            oneb[pl.ds(16 * v, 16)] = jnp.full((16,), 1.0, jnp.float32)

        plsc.subcore_barrier()

        @pl.loop(0, NROUND)
        def _(r):
            load_round(r)

            @pl.loop(0, RB)
            def _(jb):
                pltpu.async_copy(oneb, deg_sh.at[rowp.at[jb]], dsem, add=True)

            @pl.loop(0, RB)
            def _(jb):
                pltpu.make_async_copy(oneb, deg_sh.at[rowp.at[jb]],
                                      dsem).wait()

        @pl.when(s < NB_REM)
        def _():
            pltpu.sync_copy(row_hbm.at[pl.ds(rem_b, 1)], rowb.at[pl.ds(0, 1)])
            pltpu.sync_copy(col_hbm.at[pl.ds(rem_b, 1)], colb.at[pl.ds(0, 1)])
            for v in range(B // 16):
                sl = pl.ds(16 * v, 16)
                rv = rowb[0, sl]
                cv = colb[0, sl]
                nonself = rv != cv
                rowb[0, sl] = jnp.where(nonself, rv, N)
                colb[0, sl] = jnp.where(nonself, cv, N)
            pltpu.sync_copy(oneb, deg_sh.at[rowb.at[0]], add=True)

        plsc.subcore_barrier()

        pltpu.sync_copy(deg_sh.at[pl.ds(s * DSLICE, DSLICE)], disb)

        @pl.loop(0, DSLICE // 16)
        def _(j):
            sl = pl.ds(16 * j, 16)
            disb[sl] = _rsqrt16(disb[sl] + 1.0)

        pltpu.sync_copy(disb, deg_sh.at[pl.ds(s * DSLICE, DSLICE)])
        plsc.subcore_barrier()

        nch_s = (NCHUNK_FULL // NSUB) + jnp.where(s < NCHUNK_FULL % NSUB, 1, 0)

        def scaled_rows(h_src, base, nrows, dst):
            pltpu.sync_copy(h_src.at[pl.ds(base, nrows)],
                            dst.at[0].at[pl.ds(0, nrows)])
            pltpu.sync_copy(deg_sh.at[pl.ds(base, ROW_CHUNK + 16)], dchunk)

            @pl.loop(0, nrows)
            def _(r):
                d = dchunk[pl.ds(r, 16)][0]
                for v in range(VECS):
                    sl = pl.ds(16 * v, 16)
                    dst[0, r, sl] = dst[0, r, sl] * d

        def chunk_pass(h_hbm, hp_hbm, out_idx):
            def scale_chunk(base, nrows):
                scaled_rows(h_hbm, base, nrows, gbuf)
                pltpu.sync_copy(gbuf.at[0].at[pl.ds(0, nrows)],
                                hp_hbm.at[pl.ds(base, nrows)])
                pltpu.sync_copy(gbuf.at[0].at[pl.ds(0, nrows)],
                                out_sh.at[pl.ds(base, nrows)])

            @pl.loop(0, nch_s)
            def _(t):
                base = pl.multiple_of(ROW_CHUNK * (s + NSUB * t), ROW_CHUNK)
                scale_chunk(base, ROW_CHUNK)

            @pl.when(s == NSUB - 1)
            def _():
                scale_chunk(TAIL_BASE, TAIL_ROWS)

            plsc.subcore_barrier()

            @pl.loop(0, NROUND)
            def _(r):
                load_round(r)
                pltpu.async_copy(hp_hbm.at[rowp.at[0]], gbuf.at[0], gsem)

                @pl.loop(0, RB, step=2)
                def _(jb):
                    for bslot in range(2):
                        j = jb + bslot
                        nxt = j + 1

                        @pl.when(nxt < RB)
                        def _():
                            pltpu.async_copy(hp_hbm.at[rowp.at[nxt]],
                                             gbuf.at[1 - bslot], gsem)

                        pltpu.make_async_copy(hp_hbm.at[rowp.at[j]],
                                              gbuf.at[bslot], gsem).wait()
                        pltpu.sync_copy(gbuf.at[bslot],
                                        out_sh.at[colp.at[j]], add=True)

            @pl.when(s < NB_REM)
            def _():
                pltpu.sync_copy(hp_hbm.at[rowb.at[0]], gbuf.at[0])
                pltpu.sync_copy(gbuf.at[0], out_sh.at[colb.at[0]], add=True)

            plsc.subcore_barrier()

            def writeout_chunk(base, nrows):
                scaled_rows(out_sh, base, nrows, gbuf)
                pltpu.sync_copy(gbuf.at[0].at[pl.ds(0, nrows)],
                                out_hbm.at[out_idx].at[pl.ds(base, nrows)])

            @pl.loop(0, nch_s)
            def _(t):
                base = pl.multiple_of(ROW_CHUNK * (s + NSUB * t), ROW_CHUNK)
                writeout_chunk(base, ROW_CHUNK)

            @pl.when(s == NSUB - 1)
            def _():
                writeout_chunk(TAIL_BASE, TAIL_ROWS)

        @pl.when(c == 0)
        def _():
            chunk_pass(h0_hbm, hp0_hbm, 0)
            plsc.subcore_barrier()
            chunk_pass(h1_hbm, hp1_hbm, 1)

        @pl.when(c == 1)
        def _():
            chunk_pass(h2_hbm, hp2_hbm, 2)

    return k(row2d, col2d, h0, h1, h2)


def kernel(x, edge_index, W, b):
    w_pad = jnp.pad(W, ((0, 0), (0, OUT_PAD - OUT)))
    b_pad = jnp.pad(b, (0, OUT_PAD - OUT)).reshape(1, OUT_PAD)
    h0, h1, h2 = _tc_linear(x, w_pad, b_pad)
    row2d = edge_index[0].astype(jnp.int32).reshape(NB, B)
    col2d = edge_index[1].astype(jnp.int32).reshape(NB, B)
    out3, _, _, _ = _sc_gcn(row2d, col2d, h0, h1, h2)
    return jnp.concatenate([out3[0], out3[1], out3[2]], axis=1)[:, :OUT]


# speedup vs baseline: 14.2083x; 1.0788x over previous
import functools

import jax
import jax.numpy as jnp
from jax import lax
from jax.experimental import pallas as pl
from jax.experimental.pallas import tpu as pltpu
from jax.experimental.pallas import tpu_sc as plsc

N = 10000
L = 8
D = 128
OUT = 300
CHUNK = 128
NCHUNK = 3
OUT_PAD = CHUNK * NCHUNK
E = 320000

B = 128
NB = E // B
NSUB = 16
NBQ = NB // NSUB
RB = 52
NROUND = NBQ // RB
NB_REM = NB % NSUB
NPAD = 10240
NACC = N + 8
NHP = N + 16
ROW_CHUNK = 128
NCHUNK_FULL = N // ROW_CHUNK
TAIL_BASE = NCHUNK_FULL * ROW_CHUNK
TAIL_ROWS = N - TAIL_BASE
VECS = CHUNK // 16
DSLICE = NPAD // NSUB


def _tc_linear(x, w_pad, b_pad):
    blk = 400

    def body(x_ref, w_ref, b_ref, h0_ref, h1_ref, h2_ref):
        xm = jnp.mean(x_ref[...], axis=1)
        h = jnp.dot(xm, w_ref[...], preferred_element_type=jnp.float32)
        h = h + b_ref[...]
        h0_ref[...] = h[:, :CHUNK]
        h1_ref[...] = h[:, CHUNK:2 * CHUNK]
        h2_ref[...] = h[:, 2 * CHUNK:]

    return pl.pallas_call(
        body,
        grid=(N // blk,),
        in_specs=[
            pl.BlockSpec((blk, L, D), lambda i: (i, 0, 0)),
            pl.BlockSpec((D, OUT_PAD), lambda i: (0, 0)),
            pl.BlockSpec((1, OUT_PAD), lambda i: (0, 0)),
        ],
        out_specs=[
            pl.BlockSpec((blk, CHUNK), lambda i: (i, 0)),
            pl.BlockSpec((blk, CHUNK), lambda i: (i, 0)),
            pl.BlockSpec((blk, CHUNK), lambda i: (i, 0)),
        ],
        out_shape=[
            jax.ShapeDtypeStruct((N, CHUNK), jnp.float32),
            jax.ShapeDtypeStruct((N, CHUNK), jnp.float32),
            jax.ShapeDtypeStruct((N, CHUNK), jnp.float32),
        ],
    )(x, w_pad, b_pad)


def _tc_pack(out3):
    blk = 400

    def body(p_ref, o_ref):
        o_ref[:, :CHUNK] = p_ref[0]
        o_ref[:, CHUNK:2 * CHUNK] = p_ref[1]
        o_ref[:, 2 * CHUNK:] = p_ref[2][:, :OUT - 2 * CHUNK]

    return pl.pallas_call(
        body,
        grid=(N // blk,),
        in_specs=[pl.BlockSpec((NCHUNK, blk, CHUNK), lambda i: (0, i, 0))],
        out_specs=pl.BlockSpec((blk, OUT), lambda i: (i, 0)),
        out_shape=jax.ShapeDtypeStruct((N, OUT), jnp.float32),
    )(out3)


def _rsqrt16(v):
    i = lax.bitcast_convert_type(v, jnp.int32)
    i = jnp.int32(0x5F3759DF) - (i >> 1)
    y = lax.bitcast_convert_type(i, jnp.float32)
    for _ in range(3):
        y = y * (1.5 - 0.5 * v * y * y)
    return y


def _sc_gcn(row2d, col2d, h0, h1, h2):
    mesh = plsc.VectorSubcoreMesh(core_axis_name="c", subcore_axis_name="s")

    @functools.partial(
        pl.kernel,
        out_type=(
            jax.ShapeDtypeStruct((NCHUNK, N, CHUNK), jnp.float32),
            jax.ShapeDtypeStruct((NHP, CHUNK), jnp.float32),
            jax.ShapeDtypeStruct((NHP, CHUNK), jnp.float32),
            jax.ShapeDtypeStruct((NHP, CHUNK), jnp.float32),
        ),
        mesh=mesh,
        compiler_params=pltpu.CompilerParams(
            needs_layout_passes=False, use_tc_tiling_on_sc=False),
        scratch_types=dict(
            deg_sh=pltpu.VMEM_SHARED((NPAD,), jnp.float32),
            out_sh=pltpu.VMEM_SHARED((NACC, CHUNK), jnp.float32),
            disb=pltpu.VMEM((DSLICE,), jnp.float32),
            dchunk=pltpu.VMEM((ROW_CHUNK + 16,), jnp.float32),
            rowp=pltpu.VMEM((RB, B), jnp.int32),
            colp=pltpu.VMEM((RB, B), jnp.int32),
            rowb=pltpu.VMEM((2, B), jnp.int32),
            colb=pltpu.VMEM((2, B), jnp.int32),
            oneb=pltpu.VMEM((B,), jnp.float32),
            gbuf=pltpu.VMEM((2, B, CHUNK), jnp.float32),
            gsem=pltpu.SemaphoreType.DMA,
            dsem=pltpu.SemaphoreType.DMA,
        ),
    )
    def k(row_hbm, col_hbm, h0_hbm, h1_hbm, h2_hbm,
          out_hbm, hp0_hbm, hp1_hbm, hp2_hbm,
          deg_sh, out_sh, disb, dchunk, rowp, colp, rowb, colb, oneb, gbuf,
          gsem, dsem):
        c = lax.axis_index("c")
        s = lax.axis_index("s")
        b0 = NBQ * s
        rem_b = NB - NB_REM + s

        def load_round(r):
            pltpu.sync_copy(row_hbm.at[pl.ds(b0 + r * RB, RB)], rowp)
            pltpu.sync_copy(col_hbm.at[pl.ds(b0 + r * RB, RB)], colp)

            @pl.loop(0, RB)
            def _(jb):
                for v in range(B // 16):
                    sl = pl.ds(16 * v, 16)
                    rv = rowp[jb, sl]
                    cv = colp[jb, sl]
                    nonself = rv != cv
                    rowp[jb, sl] = jnp.where(nonself, rv, N)
                    colp[jb, sl] = jnp.where(nonself, cv, N)

        @pl.loop(0, DSLICE // 16)
        def _(j):
            disb[pl.ds(16 * j, 16)] = jnp.zeros((16,), jnp.float32)

        pltpu.sync_copy(disb, deg_sh.at[pl.ds(s * DSLICE, DSLICE)])
        for v in range(B // 16):
            oneb[pl.ds(16 * v, 16)] = jnp.full((16,), 1.0, jnp.float32)

        plsc.subcore_barrier()

        @pl.loop(0, NROUND)
        def _(r):
            load_round(r)

            @pl.loop(0, RB)
            def _(jb):
                pltpu.async_copy(oneb, deg_sh.at[rowp.at[jb]], dsem, add=True)

            @pl.loop(0, RB)
            def _(jb):
                pltpu.make_async_copy(oneb, deg_sh.at[rowp.at[jb]],
                                      dsem).wait()

        @pl.when(s < NB_REM)
        def _():
            pltpu.sync_copy(row_hbm.at[pl.ds(rem_b, 1)], rowb.at[pl.ds(0, 1)])
            pltpu.sync_copy(col_hbm.at[pl.ds(rem_b, 1)], colb.at[pl.ds(0, 1)])
            for v in range(B // 16):
                sl = pl.ds(16 * v, 16)
                rv = rowb[0, sl]
                cv = colb[0, sl]
                nonself = rv != cv
                rowb[0, sl] = jnp.where(nonself, rv, N)
                colb[0, sl] = jnp.where(nonself, cv, N)
            pltpu.sync_copy(oneb, deg_sh.at[rowb.at[0]], add=True)

        plsc.subcore_barrier()

        pltpu.sync_copy(deg_sh.at[pl.ds(s * DSLICE, DSLICE)], disb)

        @pl.loop(0, DSLICE // 16)
        def _(j):
            sl = pl.ds(16 * j, 16)
            disb[sl] = _rsqrt16(disb[sl] + 1.0)

        pltpu.sync_copy(disb, deg_sh.at[pl.ds(s * DSLICE, DSLICE)])
        plsc.subcore_barrier()

        nch_s = (NCHUNK_FULL // NSUB) + jnp.where(s < NCHUNK_FULL % NSUB, 1, 0)

        def scaled_rows(h_src, base, nrows, dst):
            pltpu.sync_copy(h_src.at[pl.ds(base, nrows)],
                            dst.at[0].at[pl.ds(0, nrows)])
            pltpu.sync_copy(deg_sh.at[pl.ds(base, ROW_CHUNK + 16)], dchunk)

            @pl.loop(0, nrows)
            def _(r):
                d = dchunk[pl.ds(r, 16)][0]
                for v in range(VECS):
                    sl = pl.ds(16 * v, 16)
                    dst[0, r, sl] = dst[0, r, sl] * d

        def chunk_pass(h_hbm, hp_hbm, out_idx):
            def scale_chunk(base, nrows):
                scaled_rows(h_hbm, base, nrows, gbuf)
                pltpu.sync_copy(gbuf.at[0].at[pl.ds(0, nrows)],
                                hp_hbm.at[pl.ds(base, nrows)])
                pltpu.sync_copy(gbuf.at[0].at[pl.ds(0, nrows)],
                                out_sh.at[pl.ds(base, nrows)])

            @pl.loop(0, nch_s)
            def _(t):
                base = pl.multiple_of(ROW_CHUNK * (s + NSUB * t), ROW_CHUNK)
                scale_chunk(base, ROW_CHUNK)

            @pl.when(s == NSUB - 1)
            def _():
                scale_chunk(TAIL_BASE, TAIL_ROWS)

            plsc.subcore_barrier()

            @pl.loop(0, NROUND)
            def _(r):
                load_round(r)
                pltpu.async_copy(hp_hbm.at[rowp.at[0]], gbuf.at[0], gsem)

                @pl.loop(0, RB, step=2)
                def _(jb):
                    for bslot in range(2):
                        j = jb + bslot
                        nxt = j + 1

                        @pl.when(nxt < RB)
                        def _():
                            pltpu.async_copy(hp_hbm.at[rowp.at[nxt]],
                                             gbuf.at[1 - bslot], gsem)

                        pltpu.make_async_copy(hp_hbm.at[rowp.at[j]],
                                              gbuf.at[bslot], gsem).wait()
                        pltpu.sync_copy(gbuf.at[bslot],
                                        out_sh.at[colp.at[j]], add=True)

            @pl.when(s < NB_REM)
            def _():
                pltpu.sync_copy(hp_hbm.at[rowb.at[0]], gbuf.at[0])
                pltpu.sync_copy(gbuf.at[0], out_sh.at[colb.at[0]], add=True)

            plsc.subcore_barrier()

            def writeout_chunk(base, nrows):
                scaled_rows(out_sh, base, nrows, gbuf)
                pltpu.sync_copy(gbuf.at[0].at[pl.ds(0, nrows)],
                                out_hbm.at[out_idx].at[pl.ds(base, nrows)])

            @pl.loop(0, nch_s)
            def _(t):
                base = pl.multiple_of(ROW_CHUNK * (s + NSUB * t), ROW_CHUNK)
                writeout_chunk(base, ROW_CHUNK)

            @pl.when(s == NSUB - 1)
            def _():
                writeout_chunk(TAIL_BASE, TAIL_ROWS)

        @pl.when(c == 0)
        def _():
            chunk_pass(h0_hbm, hp0_hbm, 0)
            plsc.subcore_barrier()
            chunk_pass(h1_hbm, hp1_hbm, 1)

        @pl.when(c == 1)
        def _():
            chunk_pass(h2_hbm, hp2_hbm, 2)

    return k(row2d, col2d, h0, h1, h2)


def kernel(x, edge_index, W, b):
    w_pad = jnp.pad(W, ((0, 0), (0, OUT_PAD - OUT)))
    b_pad = jnp.pad(b, (0, OUT_PAD - OUT)).reshape(1, OUT_PAD)
    h0, h1, h2 = _tc_linear(x, w_pad, b_pad)
    row2d = edge_index[0].astype(jnp.int32).reshape(NB, B)
    col2d = edge_index[1].astype(jnp.int32).reshape(NB, B)
    out3, _, _, _ = _sc_gcn(row2d, col2d, h0, h1, h2)
    return _tc_pack(out3)


# speedup vs baseline: 16.2516x; 1.1438x over previous
import functools

import jax
import jax.numpy as jnp
from jax import lax
from jax.experimental import pallas as pl
from jax.experimental.pallas import tpu as pltpu
from jax.experimental.pallas import tpu_sc as plsc

N = 10000
L = 8
D = 128
OUT = 300
CHUNK = 128
NCHUNK = 3
OUT_PAD = CHUNK * NCHUNK
E = 320000

B = 128
NB = E // B
NSUB = 16
NBQ = NB // NSUB
NB_REM = NB % NSUB
NBH = NB // 2
NBQH = NBH // NSUB
NBH_REM = NBH % NSUB
RB = 26
NPAD = 10240
NACC = N + 8
NHP = N + 16
ROW_CHUNK = 128
NCHUNK_FULL = N // ROW_CHUNK
TAIL_BASE = NCHUNK_FULL * ROW_CHUNK
TAIL_ROWS = N - TAIL_BASE
VECS = CHUNK // 16
DSLICE = NPAD // NSUB


def _tc_linear(x, w_pad, b_pad):
    blk = 400

    def body(x_ref, w_ref, b_ref, h0_ref, h1_ref, h2_ref):
        xm = jnp.mean(x_ref[...], axis=1)
        h = jnp.dot(xm, w_ref[...], preferred_element_type=jnp.float32)
        h = h + b_ref[...]
        h0_ref[...] = h[:, :CHUNK]
        h1_ref[...] = h[:, CHUNK:2 * CHUNK]
        h2_ref[...] = h[:, 2 * CHUNK:]

    return pl.pallas_call(
        body,
        grid=(N // blk,),
        in_specs=[
            pl.BlockSpec((blk, L, D), lambda i: (i, 0, 0)),
            pl.BlockSpec((D, OUT_PAD), lambda i: (0, 0)),
            pl.BlockSpec((1, OUT_PAD), lambda i: (0, 0)),
        ],
        out_specs=[
            pl.BlockSpec((blk, CHUNK), lambda i: (i, 0)),
            pl.BlockSpec((blk, CHUNK), lambda i: (i, 0)),
            pl.BlockSpec((blk, CHUNK), lambda i: (i, 0)),
        ],
        out_shape=[
            jax.ShapeDtypeStruct((N, CHUNK), jnp.float32),
            jax.ShapeDtypeStruct((N, CHUNK), jnp.float32),
            jax.ShapeDtypeStruct((N, CHUNK), jnp.float32),
        ],
    )(x, w_pad, b_pad)


def _tc_pack(out4):
    blk = 400

    def body(p_ref, o_ref):
        o_ref[:, :CHUNK] = p_ref[0]
        o_ref[:, CHUNK:2 * CHUNK] = p_ref[1] + p_ref[2]
        o_ref[:, 2 * CHUNK:] = p_ref[3][:, :OUT - 2 * CHUNK]

    return pl.pallas_call(
        body,
        grid=(N // blk,),
        in_specs=[pl.BlockSpec((4, blk, CHUNK), lambda i: (0, i, 0))],
        out_specs=pl.BlockSpec((blk, OUT), lambda i: (i, 0)),
        out_shape=jax.ShapeDtypeStruct((N, OUT), jnp.float32),
    )(out4)


def _rsqrt16(v):
    i = lax.bitcast_convert_type(v, jnp.int32)
    i = jnp.int32(0x5F3759DF) - (i >> 1)
    y = lax.bitcast_convert_type(i, jnp.float32)
    for _ in range(3):
        y = y * (1.5 - 0.5 * v * y * y)
    return y


def _sc_gcn(row2d, col2d, h0, h1, h2):
    mesh = plsc.VectorSubcoreMesh(core_axis_name="c", subcore_axis_name="s")

    @functools.partial(
        pl.kernel,
        out_type=(
            jax.ShapeDtypeStruct((4, N, CHUNK), jnp.float32),
            jax.ShapeDtypeStruct((NHP, CHUNK), jnp.float32),
            jax.ShapeDtypeStruct((NHP, CHUNK), jnp.float32),
            jax.ShapeDtypeStruct((NHP, CHUNK), jnp.float32),
            jax.ShapeDtypeStruct((NHP, CHUNK), jnp.float32),
        ),
        mesh=mesh,
        compiler_params=pltpu.CompilerParams(
            needs_layout_passes=False, use_tc_tiling_on_sc=False),
        scratch_types=dict(
            deg_sh=pltpu.VMEM_SHARED((NPAD,), jnp.float32),
            out_sh=pltpu.VMEM_SHARED((NACC, CHUNK), jnp.float32),
            disb=pltpu.VMEM((DSLICE,), jnp.float32),
            dchunk=pltpu.VMEM((ROW_CHUNK + 16,), jnp.float32),
            rowp=pltpu.VMEM((RB, B), jnp.int32),
            colp=pltpu.VMEM((RB, B), jnp.int32),
            rowb=pltpu.VMEM((2, B), jnp.int32),
            colb=pltpu.VMEM((2, B), jnp.int32),
            oneb=pltpu.VMEM((B,), jnp.float32),
            gbuf=pltpu.VMEM((2, B, CHUNK), jnp.float32),
            gsem=pltpu.SemaphoreType.DMA,
            dsem=pltpu.SemaphoreType.DMA,
        ),
    )
    def k(row_hbm, col_hbm, h0_hbm, h1_hbm, h2_hbm,
          out_hbm, hp0_hbm, hp1a_hbm, hp1b_hbm, hp2_hbm,
          deg_sh, out_sh, disb, dchunk, rowp, colp, rowb, colb, oneb, gbuf,
          gsem, dsem):
        c = lax.axis_index("c")
        s = lax.axis_index("s")

        def load_round(b_start, r):
            pltpu.sync_copy(row_hbm.at[pl.ds(b_start + r * RB, RB)], rowp)
            pltpu.sync_copy(col_hbm.at[pl.ds(b_start + r * RB, RB)], colp)

            @pl.loop(0, RB)
            def _(jb):
                for v in range(B // 16):
                    sl = pl.ds(16 * v, 16)
                    rv = rowp[jb, sl]
                    cv = colp[jb, sl]
                    nonself = rv != cv
                    rowp[jb, sl] = jnp.where(nonself, rv, N)
                    colp[jb, sl] = jnp.where(nonself, cv, N)

        def load_remainder(batch_id):
            pltpu.sync_copy(row_hbm.at[pl.ds(batch_id, 1)],
                            rowb.at[pl.ds(0, 1)])
            pltpu.sync_copy(col_hbm.at[pl.ds(batch_id, 1)],
                            colb.at[pl.ds(0, 1)])
            for v in range(B // 16):
                sl = pl.ds(16 * v, 16)
                rv = rowb[0, sl]
                cv = colb[0, sl]
                nonself = rv != cv
                rowb[0, sl] = jnp.where(nonself, rv, N)
                colb[0, sl] = jnp.where(nonself, cv, N)

        @pl.loop(0, DSLICE // 16)
        def _(j):
            disb[pl.ds(16 * j, 16)] = jnp.zeros((16,), jnp.float32)

        pltpu.sync_copy(disb, deg_sh.at[pl.ds(s * DSLICE, DSLICE)])
        for v in range(B // 16):
            oneb[pl.ds(16 * v, 16)] = jnp.full((16,), 1.0, jnp.float32)

        plsc.subcore_barrier()

        @pl.loop(0, NBQ // RB)
        def _(r):
            load_round(NBQ * s, r)

            @pl.loop(0, RB)
            def _(jb):
                pltpu.async_copy(oneb, deg_sh.at[rowp.at[jb]], dsem, add=True)

            @pl.loop(0, RB)
            def _(jb):
                pltpu.make_async_copy(oneb, deg_sh.at[rowp.at[jb]],
                                      dsem).wait()

        @pl.when(s < NB_REM)
        def _():
            load_remainder(NB - NB_REM + s)
            pltpu.sync_copy(oneb, deg_sh.at[rowb.at[0]], add=True)

        plsc.subcore_barrier()

        pltpu.sync_copy(deg_sh.at[pl.ds(s * DSLICE, DSLICE)], disb)

        @pl.loop(0, DSLICE // 16)
        def _(j):
            sl = pl.ds(16 * j, 16)
            disb[sl] = _rsqrt16(disb[sl] + 1.0)

        pltpu.sync_copy(disb, deg_sh.at[pl.ds(s * DSLICE, DSLICE)])
        plsc.subcore_barrier()

        nch_s = (NCHUNK_FULL // NSUB) + jnp.where(s < NCHUNK_FULL % NSUB, 1, 0)

        def scaled_rows(h_src, base, nrows, dst):
            pltpu.sync_copy(h_src.at[pl.ds(base, nrows)],
                            dst.at[0].at[pl.ds(0, nrows)])
            pltpu.sync_copy(deg_sh.at[pl.ds(base, ROW_CHUNK + 16)], dchunk)

            @pl.loop(0, nrows)
            def _(r):
                d = dchunk[pl.ds(r, 16)][0]
                for v in range(VECS):
                    sl = pl.ds(16 * v, 16)
                    dst[0, r, sl] = dst[0, r, sl] * d

        def for_own_rows(fn):
            @pl.loop(0, nch_s)
            def _(t):
                base = pl.multiple_of(ROW_CHUNK * (s + NSUB * t), ROW_CHUNK)
                fn(base, ROW_CHUNK)

            @pl.when(s == NSUB - 1)
            def _():
                fn(TAIL_BASE, TAIL_ROWS)

        def edge_pass(h_hbm, hp_hbm, out_idx, b_base, nbq, n_rem, self_init):
            def scale_chunk(base, nrows):
                scaled_rows(h_hbm, base, nrows, gbuf)
                pltpu.sync_copy(gbuf.at[0].at[pl.ds(0, nrows)],
                                hp_hbm.at[pl.ds(base, nrows)])
                if self_init:
                    pltpu.sync_copy(gbuf.at[0].at[pl.ds(0, nrows)],
                                    out_sh.at[pl.ds(base, nrows)])

            for_own_rows(scale_chunk)

            if not self_init:
                @pl.loop(0, B)
                def _(r):
                    for v in range(VECS):
                        gbuf[1, r, pl.ds(16 * v, 16)] = (
                            jnp.zeros((16,), jnp.float32))

                def zero_chunk(base, nrows):
                    pltpu.sync_copy(gbuf.at[1].at[pl.ds(0, nrows)],
                                    out_sh.at[pl.ds(base, nrows)])

                for_own_rows(zero_chunk)

            plsc.subcore_barrier()

            @pl.loop(0, nbq // RB)
            def _(r):
                load_round(b_base + nbq * s, r)
                pltpu.async_copy(hp_hbm.at[rowp.at[0]], gbuf.at[0], gsem)

                @pl.loop(0, RB, step=2)
                def _(jb):
                    for bslot in range(2):
                        j = jb + bslot
                        nxt = j + 1

                        @pl.when(nxt < RB)
                        def _():
                            pltpu.async_copy(hp_hbm.at[rowp.at[nxt]],
                                             gbuf.at[1 - bslot], gsem)

                        pltpu.make_async_copy(hp_hbm.at[rowp.at[j]],
                                              gbuf.at[bslot], gsem).wait()
                        pltpu.sync_copy(gbuf.at[bslot],
                                        out_sh.at[colp.at[j]], add=True)

            @pl.when(s < n_rem)
            def _():
                load_remainder(b_base + NSUB * nbq + s)
                pltpu.sync_copy(hp_hbm.at[rowb.at[0]], gbuf.at[0])
                pltpu.sync_copy(gbuf.at[0], out_sh.at[colb.at[0]], add=True)

            plsc.subcore_barrier()

            def writeout_chunk(base, nrows):
                scaled_rows(out_sh, base, nrows, gbuf)
                pltpu.sync_copy(gbuf.at[0].at[pl.ds(0, nrows)],
                                out_hbm.at[out_idx].at[pl.ds(base, nrows)])

            for_own_rows(writeout_chunk)

        @pl.when(c == 0)
        def _():
            edge_pass(h0_hbm, hp0_hbm, 0, 0, NBQ, NB_REM, True)
            plsc.subcore_barrier()
            edge_pass(h1_hbm, hp1a_hbm, 1, 0, NBQH, NBH_REM, True)

        @pl.when(c == 1)
        def _():
            edge_pass(h2_hbm, hp2_hbm, 3, 0, NBQ, NB_REM, True)
            plsc.subcore_barrier()
            edge_pass(h1_hbm, hp1b_hbm, 2, NBH, NBQH, NBH_REM, False)

    return k(row2d, col2d, h0, h1, h2)


def kernel(x, edge_index, W, b):
    w_pad = jnp.pad(W, ((0, 0), (0, OUT_PAD - OUT)))
    b_pad = jnp.pad(b, (0, OUT_PAD - OUT)).reshape(1, OUT_PAD)
    h0, h1, h2 = _tc_linear(x, w_pad, b_pad)
    row2d = edge_index[0].astype(jnp.int32).reshape(NB, B)
    col2d = edge_index[1].astype(jnp.int32).reshape(NB, B)
    out4, _, _, _, _ = _sc_gcn(row2d, col2d, h0, h1, h2)
    return _tc_pack(out4)


# speedup vs baseline: 16.2758x; 1.0015x over previous
import functools

import jax
import jax.numpy as jnp
from jax import lax
from jax.experimental import pallas as pl
from jax.experimental.pallas import tpu as pltpu
from jax.experimental.pallas import tpu_sc as plsc

N = 10000
L = 8
D = 128
OUT = 300
CHUNK = 128
NCHUNK = 3
OUT_PAD = CHUNK * NCHUNK
E = 320000

B = 128
NB = E // B
NSUB = 16
NBQ = NB // NSUB
NB_REM = NB % NSUB
NBH = NB // 2
NBQH = NBH // NSUB
NBH_REM = NBH % NSUB
RB = 26
NPAD = 10240
NACC = N + 8
NHP = N + 16
ROW_CHUNK = 128
NCHUNK_FULL = N // ROW_CHUNK
TAIL_BASE = NCHUNK_FULL * ROW_CHUNK
TAIL_ROWS = N - TAIL_BASE
VECS = CHUNK // 16
DSLICE = NPAD // NSUB


def _tc_linear(x, w_pad, b_pad):
    blk = 400

    def body(x_ref, w_ref, b_ref, h0_ref, h1_ref, h2_ref):
        xm = jnp.mean(x_ref[...], axis=1)
        h = jnp.dot(xm, w_ref[...], preferred_element_type=jnp.float32)
        h = h + b_ref[...]
        h0_ref[...] = h[:, :CHUNK]
        h1_ref[...] = h[:, CHUNK:2 * CHUNK]
        h2_ref[...] = h[:, 2 * CHUNK:]

    return pl.pallas_call(
        body,
        grid=(N // blk,),
        in_specs=[
            pl.BlockSpec((blk, L, D), lambda i: (i, 0, 0)),
            pl.BlockSpec((D, OUT_PAD), lambda i: (0, 0)),
            pl.BlockSpec((1, OUT_PAD), lambda i: (0, 0)),
        ],
        out_specs=[
            pl.BlockSpec((blk, CHUNK), lambda i: (i, 0)),
            pl.BlockSpec((blk, CHUNK), lambda i: (i, 0)),
            pl.BlockSpec((blk, CHUNK), lambda i: (i, 0)),
        ],
        out_shape=[
            jax.ShapeDtypeStruct((N, CHUNK), jnp.float32),
            jax.ShapeDtypeStruct((N, CHUNK), jnp.float32),
            jax.ShapeDtypeStruct((N, CHUNK), jnp.float32),
        ],
    )(x, w_pad, b_pad)


def _tc_pack(out4):
    blk = 400

    def body(p_ref, o_ref):
        o_ref[:, :CHUNK] = p_ref[0]
        o_ref[:, CHUNK:2 * CHUNK] = p_ref[1] + p_ref[2]
        o_ref[:, 2 * CHUNK:] = p_ref[3][:, :OUT - 2 * CHUNK]

    return pl.pallas_call(
        body,
        grid=(N // blk,),
        in_specs=[pl.BlockSpec((4, blk, CHUNK), lambda i: (0, i, 0))],
        out_specs=pl.BlockSpec((blk, OUT), lambda i: (i, 0)),
        out_shape=jax.ShapeDtypeStruct((N, OUT), jnp.float32),
    )(out4)


def _rsqrt16(v):
    i = lax.bitcast_convert_type(v, jnp.int32)
    i = jnp.int32(0x5F3759DF) - (i >> 1)
    y = lax.bitcast_convert_type(i, jnp.float32)
    for _ in range(3):
        y = y * (1.5 - 0.5 * v * y * y)
    return y


def _sc_gcn(row2d, col2d, h0, h1, h2):
    mesh = plsc.VectorSubcoreMesh(core_axis_name="c", subcore_axis_name="s")

    @functools.partial(
        pl.kernel,
        out_type=(
            jax.ShapeDtypeStruct((4, N, CHUNK), jnp.float32),
            jax.ShapeDtypeStruct((NHP, CHUNK), jnp.float32),
            jax.ShapeDtypeStruct((NHP, CHUNK), jnp.float32),
            jax.ShapeDtypeStruct((NHP, CHUNK), jnp.float32),
            jax.ShapeDtypeStruct((NHP, CHUNK), jnp.float32),
        ),
        mesh=mesh,
        compiler_params=pltpu.CompilerParams(
            needs_layout_passes=False, use_tc_tiling_on_sc=False),
        scratch_types=dict(
            deg_sh=pltpu.VMEM_SHARED((NPAD,), jnp.float32),
            out_sh=pltpu.VMEM_SHARED((NACC, CHUNK), jnp.float32),
            disb=pltpu.VMEM((DSLICE,), jnp.float32),
            dchunk=pltpu.VMEM((ROW_CHUNK + 16,), jnp.float32),
            rowp=pltpu.VMEM((2, RB, B), jnp.int32),
            colp=pltpu.VMEM((2, RB, B), jnp.int32),
            rowb=pltpu.VMEM((2, B), jnp.int32),
            colb=pltpu.VMEM((2, B), jnp.int32),
            oneb=pltpu.VMEM((B,), jnp.float32),
            gbuf=pltpu.VMEM((2, B, CHUNK), jnp.float32),
            gsem=pltpu.SemaphoreType.DMA,
            dsem=pltpu.SemaphoreType.DMA,
            isem=pltpu.SemaphoreType.DMA,
        ),
    )
    def k(row_hbm, col_hbm, h0_hbm, h1_hbm, h2_hbm,
          out_hbm, hp0_hbm, hp1a_hbm, hp1b_hbm, hp2_hbm,
          deg_sh, out_sh, disb, dchunk, rowp, colp, rowb, colb, oneb, gbuf,
          gsem, dsem, isem):
        c = lax.axis_index("c")
        s = lax.axis_index("s")

        def issue_round(b_start, r, pb):
            pltpu.async_copy(row_hbm.at[pl.ds(b_start + r * RB, RB)],
                             rowp.at[pb], isem)
            pltpu.async_copy(col_hbm.at[pl.ds(b_start + r * RB, RB)],
                             colp.at[pb], isem)

        def finish_round(b_start, r, pb):
            pltpu.make_async_copy(row_hbm.at[pl.ds(b_start + r * RB, RB)],
                                  rowp.at[pb], isem).wait()
            pltpu.make_async_copy(col_hbm.at[pl.ds(b_start + r * RB, RB)],
                                  colp.at[pb], isem).wait()

            @pl.loop(0, RB)
            def _(jb):
                for v in range(B // 16):
                    sl = pl.ds(16 * v, 16)
                    rv = rowp[pb, jb, sl]
                    cv = colp[pb, jb, sl]
                    nonself = rv != cv
                    rowp[pb, jb, sl] = jnp.where(nonself, rv, N)
                    colp[pb, jb, sl] = jnp.where(nonself, cv, N)

        def load_remainder(batch_id):
            pltpu.sync_copy(row_hbm.at[pl.ds(batch_id, 1)],
                            rowb.at[pl.ds(0, 1)])
            pltpu.sync_copy(col_hbm.at[pl.ds(batch_id, 1)],
                            colb.at[pl.ds(0, 1)])
            for v in range(B // 16):
                sl = pl.ds(16 * v, 16)
                rv = rowb[0, sl]
                cv = colb[0, sl]
                nonself = rv != cv
                rowb[0, sl] = jnp.where(nonself, rv, N)
                colb[0, sl] = jnp.where(nonself, cv, N)

        @pl.loop(0, DSLICE // 16)
        def _(j):
            disb[pl.ds(16 * j, 16)] = jnp.zeros((16,), jnp.float32)

        pltpu.sync_copy(disb, deg_sh.at[pl.ds(s * DSLICE, DSLICE)])
        for v in range(B // 16):
            oneb[pl.ds(16 * v, 16)] = jnp.full((16,), 1.0, jnp.float32)

        plsc.subcore_barrier()

        issue_round(NBQ * s, 0, 0)

        @pl.loop(0, NBQ // RB)
        def _(r):
            pb = r % 2
            finish_round(NBQ * s, r, pb)

            @pl.when(r + 1 < NBQ // RB)
            def _():
                issue_round(NBQ * s, r + 1, 1 - pb)

            @pl.loop(0, RB)
            def _(jb):
                pltpu.async_copy(oneb, deg_sh.at[rowp.at[pb, jb]], dsem,
                                 add=True)

            @pl.loop(0, RB)
            def _(jb):
                pltpu.make_async_copy(oneb, deg_sh.at[rowp.at[pb, jb]],
                                      dsem).wait()

        @pl.when(s < NB_REM)
        def _():
            load_remainder(NB - NB_REM + s)
            pltpu.sync_copy(oneb, deg_sh.at[rowb.at[0]], add=True)

        plsc.subcore_barrier()

        pltpu.sync_copy(deg_sh.at[pl.ds(s * DSLICE, DSLICE)], disb)

        @pl.loop(0, DSLICE // 16)
        def _(j):
            sl = pl.ds(16 * j, 16)
            disb[sl] = _rsqrt16(disb[sl] + 1.0)

        pltpu.sync_copy(disb, deg_sh.at[pl.ds(s * DSLICE, DSLICE)])
        plsc.subcore_barrier()

        nch_s = (NCHUNK_FULL // NSUB) + jnp.where(s < NCHUNK_FULL % NSUB, 1, 0)

        def scaled_rows(h_src, base, nrows, dst):
            pltpu.sync_copy(h_src.at[pl.ds(base, nrows)],
                            dst.at[0].at[pl.ds(0, nrows)])
            pltpu.sync_copy(deg_sh.at[pl.ds(base, ROW_CHUNK + 16)], dchunk)

            @pl.loop(0, nrows)
            def _(r):
                d = dchunk[pl.ds(r, 16)][0]
                for v in range(VECS):
                    sl = pl.ds(16 * v, 16)
                    dst[0, r, sl] = dst[0, r, sl] * d

        def for_own_rows(fn):
            @pl.loop(0, nch_s)
            def _(t):
                base = pl.multiple_of(ROW_CHUNK * (s + NSUB * t), ROW_CHUNK)
                fn(base, ROW_CHUNK)

            @pl.when(s == NSUB - 1)
            def _():
                fn(TAIL_BASE, TAIL_ROWS)

        def edge_pass(h_hbm, hp_hbm, out_idx, b_base, nbq, n_rem, self_init):
            def scale_chunk(base, nrows):
                scaled_rows(h_hbm, base, nrows, gbuf)
                pltpu.sync_copy(gbuf.at[0].at[pl.ds(0, nrows)],
                                hp_hbm.at[pl.ds(base, nrows)])
                if self_init:
                    pltpu.sync_copy(gbuf.at[0].at[pl.ds(0, nrows)],
                                    out_sh.at[pl.ds(base, nrows)])

            for_own_rows(scale_chunk)

            if not self_init:
                @pl.loop(0, B)
                def _(r):
                    for v in range(VECS):
                        gbuf[1, r, pl.ds(16 * v, 16)] = (
                            jnp.zeros((16,), jnp.float32))

                def zero_chunk(base, nrows):
                    pltpu.sync_copy(gbuf.at[1].at[pl.ds(0, nrows)],
                                    out_sh.at[pl.ds(base, nrows)])

                for_own_rows(zero_chunk)

            plsc.subcore_barrier()

            issue_round(b_base + nbq * s, 0, 0)

            @pl.loop(0, nbq // RB)
            def _(r):
                pb = r % 2
                finish_round(b_base + nbq * s, r, pb)

                @pl.when(r + 1 < nbq // RB)
                def _():
                    issue_round(b_base + nbq * s, r + 1, 1 - pb)

                pltpu.async_copy(hp_hbm.at[rowp.at[pb, 0]], gbuf.at[0], gsem)

                @pl.loop(0, RB, step=2)
                def _(jb):
                    for bslot in range(2):
                        j = jb + bslot
                        nxt = j + 1

                        @pl.when(nxt < RB)
                        def _():
                            pltpu.async_copy(hp_hbm.at[rowp.at[pb, nxt]],
                                             gbuf.at[1 - bslot], gsem)

                        pltpu.make_async_copy(hp_hbm.at[rowp.at[pb, j]],
                                              gbuf.at[bslot], gsem).wait()
                        pltpu.sync_copy(gbuf.at[bslot],
                                        out_sh.at[colp.at[pb, j]], add=True)

            @pl.when(s < n_rem)
            def _():
                load_remainder(b_base + NSUB * nbq + s)
                pltpu.sync_copy(hp_hbm.at[rowb.at[0]], gbuf.at[0])
                pltpu.sync_copy(gbuf.at[0], out_sh.at[colb.at[0]], add=True)

            plsc.subcore_barrier()

            def writeout_chunk(base, nrows):
                scaled_rows(out_sh, base, nrows, gbuf)
                pltpu.sync_copy(gbuf.at[0].at[pl.ds(0, nrows)],
                                out_hbm.at[out_idx].at[pl.ds(base, nrows)])

            for_own_rows(writeout_chunk)

        @pl.when(c == 0)
        def _():
            edge_pass(h0_hbm, hp0_hbm, 0, 0, NBQ, NB_REM, True)
            plsc.subcore_barrier()
            edge_pass(h1_hbm, hp1a_hbm, 1, 0, NBQH, NBH_REM, True)

        @pl.when(c == 1)
        def _():
            edge_pass(h2_hbm, hp2_hbm, 3, 0, NBQ, NB_REM, True)
            plsc.subcore_barrier()
            edge_pass(h1_hbm, hp1b_hbm, 2, NBH, NBQH, NBH_REM, False)

    return k(row2d, col2d, h0, h1, h2)


def kernel(x, edge_index, W, b):
    w_pad = jnp.pad(W, ((0, 0), (0, OUT_PAD - OUT)))
    b_pad = jnp.pad(b, (0, OUT_PAD - OUT)).reshape(1, OUT_PAD)
    h0, h1, h2 = _tc_linear(x, w_pad, b_pad)
    row2d = edge_index[0].astype(jnp.int32).reshape(NB, B)
    col2d = edge_index[1].astype(jnp.int32).reshape(NB, B)
    out4, _, _, _, _ = _sc_gcn(row2d, col2d, h0, h1, h2)
    return _tc_pack(out4)


# speedup vs baseline: 16.3937x; 1.0072x over previous
import functools

import jax
import jax.numpy as jnp
from jax import lax
from jax.experimental import pallas as pl
from jax.experimental.pallas import tpu as pltpu
from jax.experimental.pallas import tpu_sc as plsc

N = 10000
L = 8
D = 128
OUT = 300
CHUNK = 128
NCHUNK = 3
OUT_PAD = CHUNK * NCHUNK
E = 320000

B = 128
NB = E // B
NSUB = 16
NBQ = NB // NSUB
NB_REM = NB % NSUB
NBH = NB // 2
NBQH = NBH // NSUB
NBH_REM = NBH % NSUB
RB = 26
NPAD = 10240
NACC = N + 8
NHP = N + 16
ROW_CHUNK = 128
NCHUNK_FULL = N // ROW_CHUNK
TAIL_BASE = NCHUNK_FULL * ROW_CHUNK
TAIL_ROWS = N - TAIL_BASE
VECS = CHUNK // 16
DSLICE = NPAD // NSUB


def _tc_linear(x, w_pad, b_pad):
    blk = 400

    def body(x_ref, w_ref, b_ref, h0_ref, h1_ref, h2_ref):
        xm = jnp.mean(x_ref[...], axis=1)
        h = jnp.dot(xm, w_ref[...], preferred_element_type=jnp.float32)
        h = h + b_ref[...]
        h0_ref[...] = h[:, :CHUNK]
        h1_ref[...] = h[:, CHUNK:2 * CHUNK]
        h2_ref[...] = h[:, 2 * CHUNK:]

    return pl.pallas_call(
        body,
        grid=(N // blk,),
        in_specs=[
            pl.BlockSpec((blk, L, D), lambda i: (i, 0, 0)),
            pl.BlockSpec((D, OUT_PAD), lambda i: (0, 0)),
            pl.BlockSpec((1, OUT_PAD), lambda i: (0, 0)),
        ],
        out_specs=[
            pl.BlockSpec((blk, CHUNK), lambda i: (i, 0)),
            pl.BlockSpec((blk, CHUNK), lambda i: (i, 0)),
            pl.BlockSpec((blk, CHUNK), lambda i: (i, 0)),
        ],
        out_shape=[
            jax.ShapeDtypeStruct((N, CHUNK), jnp.float32),
            jax.ShapeDtypeStruct((N, CHUNK), jnp.float32),
            jax.ShapeDtypeStruct((N, CHUNK), jnp.float32),
        ],
    )(x, w_pad, b_pad)


def _tc_pack(out4):
    blk = 400

    def body(p_ref, o_ref):
        o_ref[:, :CHUNK] = p_ref[0]
        o_ref[:, CHUNK:2 * CHUNK] = p_ref[1] + p_ref[2]
        o_ref[:, 2 * CHUNK:] = p_ref[3][:, :OUT - 2 * CHUNK]

    return pl.pallas_call(
        body,
        grid=(N // blk,),
        in_specs=[pl.BlockSpec((4, blk, CHUNK), lambda i: (0, i, 0))],
        out_specs=pl.BlockSpec((blk, OUT), lambda i: (i, 0)),
        out_shape=jax.ShapeDtypeStruct((N, OUT), jnp.float32),
    )(out4)


def _rsqrt16(v):
    i = lax.bitcast_convert_type(v, jnp.int32)
    i = jnp.int32(0x5F3759DF) - (i >> 1)
    y = lax.bitcast_convert_type(i, jnp.float32)
    for _ in range(3):
        y = y * (1.5 - 0.5 * v * y * y)
    return y


def _sc_gcn(row2d, col2d, h0, h1, h2):
    mesh = plsc.VectorSubcoreMesh(core_axis_name="c", subcore_axis_name="s")

    @functools.partial(
        pl.kernel,
        out_type=(
            jax.ShapeDtypeStruct((4, N, CHUNK), jnp.float32),
            jax.ShapeDtypeStruct((NHP, CHUNK), jnp.float32),
            jax.ShapeDtypeStruct((NHP, CHUNK), jnp.float32),
            jax.ShapeDtypeStruct((NHP, CHUNK), jnp.float32),
            jax.ShapeDtypeStruct((NHP, CHUNK), jnp.float32),
        ),
        mesh=mesh,
        compiler_params=pltpu.CompilerParams(
            needs_layout_passes=False, use_tc_tiling_on_sc=False),
        scratch_types=dict(
            deg_sh=pltpu.VMEM_SHARED((NPAD,), jnp.float32),
            out_sh=pltpu.VMEM_SHARED((NACC, CHUNK), jnp.float32),
            disb=pltpu.VMEM((DSLICE,), jnp.float32),
            dchunk=pltpu.VMEM((ROW_CHUNK + 16,), jnp.float32),
            rowp=pltpu.VMEM((2, RB, B), jnp.int32),
            colp=pltpu.VMEM((2, RB, B), jnp.int32),
            rowb=pltpu.VMEM((2, B), jnp.int32),
            colb=pltpu.VMEM((2, B), jnp.int32),
            oneb=pltpu.VMEM((B,), jnp.float32),
            gbuf=pltpu.VMEM((2, B, CHUNK), jnp.float32),
            gsem=pltpu.SemaphoreType.DMA,
            dsem=pltpu.SemaphoreType.DMA,
            isem=pltpu.SemaphoreType.DMA,
            ssem=pltpu.SemaphoreType.DMA,
        ),
    )
    def k(row_hbm, col_hbm, h0_hbm, h1_hbm, h2_hbm,
          out_hbm, hp0_hbm, hp1a_hbm, hp1b_hbm, hp2_hbm,
          deg_sh, out_sh, disb, dchunk, rowp, colp, rowb, colb, oneb, gbuf,
          gsem, dsem, isem, ssem):
        c = lax.axis_index("c")
        s = lax.axis_index("s")

        def issue_round(b_start, r, pb):
            pltpu.async_copy(row_hbm.at[pl.ds(b_start + r * RB, RB)],
                             rowp.at[pb], isem)
            pltpu.async_copy(col_hbm.at[pl.ds(b_start + r * RB, RB)],
                             colp.at[pb], isem)

        def finish_round(b_start, r, pb):
            pltpu.make_async_copy(row_hbm.at[pl.ds(b_start + r * RB, RB)],
                                  rowp.at[pb], isem).wait()
            pltpu.make_async_copy(col_hbm.at[pl.ds(b_start + r * RB, RB)],
                                  colp.at[pb], isem).wait()

            @pl.loop(0, RB)
            def _(jb):
                for v in range(B // 16):
                    sl = pl.ds(16 * v, 16)
                    rv = rowp[pb, jb, sl]
                    cv = colp[pb, jb, sl]
                    nonself = rv != cv
                    rowp[pb, jb, sl] = jnp.where(nonself, rv, N)
                    colp[pb, jb, sl] = jnp.where(nonself, cv, N)

        def load_remainder(batch_id):
            pltpu.sync_copy(row_hbm.at[pl.ds(batch_id, 1)],
                            rowb.at[pl.ds(0, 1)])
            pltpu.sync_copy(col_hbm.at[pl.ds(batch_id, 1)],
                            colb.at[pl.ds(0, 1)])
            for v in range(B // 16):
                sl = pl.ds(16 * v, 16)
                rv = rowb[0, sl]
                cv = colb[0, sl]
                nonself = rv != cv
                rowb[0, sl] = jnp.where(nonself, rv, N)
                colb[0, sl] = jnp.where(nonself, cv, N)

        @pl.loop(0, DSLICE // 16)
        def _(j):
            disb[pl.ds(16 * j, 16)] = jnp.zeros((16,), jnp.float32)

        pltpu.sync_copy(disb, deg_sh.at[pl.ds(s * DSLICE, DSLICE)])
        for v in range(B // 16):
            oneb[pl.ds(16 * v, 16)] = jnp.full((16,), 1.0, jnp.float32)

        plsc.subcore_barrier()

        issue_round(NBQ * s, 0, 0)

        @pl.loop(0, NBQ // RB)
        def _(r):
            pb = r % 2
            finish_round(NBQ * s, r, pb)

            @pl.when(r + 1 < NBQ // RB)
            def _():
                issue_round(NBQ * s, r + 1, 1 - pb)

            @pl.loop(0, RB)
            def _(jb):
                pltpu.async_copy(oneb, deg_sh.at[rowp.at[pb, jb]], dsem,
                                 add=True)

            @pl.loop(0, RB)
            def _(jb):
                pltpu.make_async_copy(oneb, deg_sh.at[rowp.at[pb, jb]],
                                      dsem).wait()

        @pl.when(s < NB_REM)
        def _():
            load_remainder(NB - NB_REM + s)
            pltpu.sync_copy(oneb, deg_sh.at[rowb.at[0]], add=True)

        plsc.subcore_barrier()

        pltpu.sync_copy(deg_sh.at[pl.ds(s * DSLICE, DSLICE)], disb)

        @pl.loop(0, DSLICE // 16)
        def _(j):
            sl = pl.ds(16 * j, 16)
            disb[sl] = _rsqrt16(disb[sl] + 1.0)

        pltpu.sync_copy(disb, deg_sh.at[pl.ds(s * DSLICE, DSLICE)])
        plsc.subcore_barrier()

        nch_s = (NCHUNK_FULL // NSUB) + jnp.where(s < NCHUNK_FULL % NSUB, 1, 0)

        def scaled_rows(h_src, base, nrows, dst):
            pltpu.sync_copy(h_src.at[pl.ds(base, nrows)],
                            dst.at[0].at[pl.ds(0, nrows)])
            pltpu.sync_copy(deg_sh.at[pl.ds(base, ROW_CHUNK + 16)], dchunk)

            @pl.loop(0, nrows)
            def _(r):
                d = dchunk[pl.ds(r, 16)][0]
                for v in range(VECS):
                    sl = pl.ds(16 * v, 16)
                    dst[0, r, sl] = dst[0, r, sl] * d

        def for_own_rows(fn):
            @pl.loop(0, nch_s)
            def _(t):
                base = pl.multiple_of(ROW_CHUNK * (s + NSUB * t), ROW_CHUNK)
                fn(base, ROW_CHUNK)

            @pl.when(s == NSUB - 1)
            def _():
                fn(TAIL_BASE, TAIL_ROWS)

        def edge_pass(h_hbm, hp_hbm, out_idx, b_base, nbq, n_rem, self_init):
            def scale_chunk(base, nrows):
                scaled_rows(h_hbm, base, nrows, gbuf)
                pltpu.sync_copy(gbuf.at[0].at[pl.ds(0, nrows)],
                                hp_hbm.at[pl.ds(base, nrows)])
                if self_init:
                    pltpu.sync_copy(gbuf.at[0].at[pl.ds(0, nrows)],
                                    out_sh.at[pl.ds(base, nrows)])

            for_own_rows(scale_chunk)

            if not self_init:
                @pl.loop(0, B)
                def _(r):
                    for v in range(VECS):
                        gbuf[1, r, pl.ds(16 * v, 16)] = (
                            jnp.zeros((16,), jnp.float32))

                def zero_chunk(base, nrows):
                    pltpu.sync_copy(gbuf.at[1].at[pl.ds(0, nrows)],
                                    out_sh.at[pl.ds(base, nrows)])

                for_own_rows(zero_chunk)

            plsc.subcore_barrier()

            issue_round(b_base + nbq * s, 0, 0)

            @pl.loop(0, nbq // RB)
            def _(r):
                pb = r % 2
                finish_round(b_base + nbq * s, r, pb)

                @pl.when(r + 1 < nbq // RB)
                def _():
                    issue_round(b_base + nbq * s, r + 1, 1 - pb)

                pltpu.async_copy(hp_hbm.at[rowp.at[pb, 0]], gbuf.at[0], gsem)

                @pl.loop(0, RB, step=2)
                def _(jb):
                    for bslot in range(2):
                        j = jb + bslot
                        nxt = j + 1

                        @pl.when(r * RB + j >= 1)
                        def _():
                            pltpu.make_async_copy(
                                gbuf.at[1 - bslot],
                                out_sh.at[colp.at[pb, 0]], ssem).wait()

                        @pl.when(nxt < RB)
                        def _():
                            pltpu.async_copy(hp_hbm.at[rowp.at[pb, nxt]],
                                             gbuf.at[1 - bslot], gsem)

                        pltpu.make_async_copy(hp_hbm.at[rowp.at[pb, j]],
                                              gbuf.at[bslot], gsem).wait()
                        pltpu.async_copy(gbuf.at[bslot],
                                         out_sh.at[colp.at[pb, j]], ssem,
                                         add=True)

            pltpu.make_async_copy(gbuf.at[0], out_sh.at[colp.at[0, 0]],
                                  ssem).wait()

            @pl.when(s < n_rem)
            def _():
                load_remainder(b_base + NSUB * nbq + s)
                pltpu.sync_copy(hp_hbm.at[rowb.at[0]], gbuf.at[0])
                pltpu.sync_copy(gbuf.at[0], out_sh.at[colb.at[0]], add=True)

            plsc.subcore_barrier()

            def writeout_chunk(base, nrows):
                scaled_rows(out_sh, base, nrows, gbuf)
                pltpu.sync_copy(gbuf.at[0].at[pl.ds(0, nrows)],
                                out_hbm.at[out_idx].at[pl.ds(base, nrows)])

            for_own_rows(writeout_chunk)

        @pl.when(c == 0)
        def _():
            edge_pass(h0_hbm, hp0_hbm, 0, 0, NBQ, NB_REM, True)
            plsc.subcore_barrier()
            edge_pass(h1_hbm, hp1a_hbm, 1, 0, NBQH, NBH_REM, True)

        @pl.when(c == 1)
        def _():
            edge_pass(h2_hbm, hp2_hbm, 3, 0, NBQ, NB_REM, True)
            plsc.subcore_barrier()
            edge_pass(h1_hbm, hp1b_hbm, 2, NBH, NBQH, NBH_REM, False)

    return k(row2d, col2d, h0, h1, h2)


def kernel(x, edge_index, W, b):
    w_pad = jnp.pad(W, ((0, 0), (0, OUT_PAD - OUT)))
    b_pad = jnp.pad(b, (0, OUT_PAD - OUT)).reshape(1, OUT_PAD)
    h0, h1, h2 = _tc_linear(x, w_pad, b_pad)
    row2d = edge_index[0].astype(jnp.int32).reshape(NB, B)
    col2d = edge_index[1].astype(jnp.int32).reshape(NB, B)
    out4, _, _, _, _ = _sc_gcn(row2d, col2d, h0, h1, h2)
    return _tc_pack(out4)


# speedup vs baseline: 17.0313x; 1.0389x over previous
import functools

import jax
import jax.numpy as jnp
from jax import lax
from jax.experimental import pallas as pl
from jax.experimental.pallas import tpu as pltpu
from jax.experimental.pallas import tpu_sc as plsc

N = 10000
L = 8
D = 128
OUT = 300
CHUNK = 128
NCHUNK = 3
OUT_PAD = CHUNK * NCHUNK
E = 320000

B = 128
NB = E // B
NSUB = 16
NBQ = NB // NSUB
NB_REM = NB % NSUB
NBH = NB // 2
NBQH = NBH // NSUB
NBH_REM = NBH % NSUB
RB = 26
NPAD = 10240
NACC = N + 8
NHP = N + 16
ROW_CHUNK = 128
NCHUNK_FULL = N // ROW_CHUNK
TAIL_BASE = NCHUNK_FULL * ROW_CHUNK
TAIL_ROWS = N - TAIL_BASE
VECS = CHUNK // 16
DSLICE = NPAD // NSUB


def _tc_linear(x, w_pad, b_pad):
    blk = 1000

    def body(x_ref, w_ref, b_ref, h0_ref, h1_ref, h2_ref):
        xm = jnp.mean(x_ref[...], axis=1)
        h = jnp.dot(xm, w_ref[...], preferred_element_type=jnp.float32)
        h = h + b_ref[...]
        h0_ref[...] = h[:, :CHUNK]
        h1_ref[...] = h[:, CHUNK:2 * CHUNK]
        h2_ref[...] = h[:, 2 * CHUNK:]

    return pl.pallas_call(
        body,
        grid=(N // blk,),
        in_specs=[
            pl.BlockSpec((blk, L, D), lambda i: (i, 0, 0)),
            pl.BlockSpec((D, OUT_PAD), lambda i: (0, 0)),
            pl.BlockSpec((1, OUT_PAD), lambda i: (0, 0)),
        ],
        out_specs=[
            pl.BlockSpec((blk, CHUNK), lambda i: (i, 0)),
            pl.BlockSpec((blk, CHUNK), lambda i: (i, 0)),
            pl.BlockSpec((blk, CHUNK), lambda i: (i, 0)),
        ],
        out_shape=[
            jax.ShapeDtypeStruct((N, CHUNK), jnp.float32),
            jax.ShapeDtypeStruct((N, CHUNK), jnp.float32),
            jax.ShapeDtypeStruct((N, CHUNK), jnp.float32),
        ],
    )(x, w_pad, b_pad)


def _tc_pack(out4):
    blk = 2000

    def body(p_ref, o_ref):
        o_ref[:, :CHUNK] = p_ref[0]
        o_ref[:, CHUNK:2 * CHUNK] = p_ref[1] + p_ref[2]
        o_ref[:, 2 * CHUNK:] = p_ref[3][:, :OUT - 2 * CHUNK]

    return pl.pallas_call(
        body,
        grid=(N // blk,),
        in_specs=[pl.BlockSpec((4, blk, CHUNK), lambda i: (0, i, 0))],
        out_specs=pl.BlockSpec((blk, OUT), lambda i: (i, 0)),
        out_shape=jax.ShapeDtypeStruct((N, OUT), jnp.float32),
    )(out4)


def _rsqrt16(v):
    i = lax.bitcast_convert_type(v, jnp.int32)
    i = jnp.int32(0x5F3759DF) - (i >> 1)
    y = lax.bitcast_convert_type(i, jnp.float32)
    for _ in range(3):
        y = y * (1.5 - 0.5 * v * y * y)
    return y


def _sc_gcn(row2d, col2d, h0, h1, h2):
    mesh = plsc.VectorSubcoreMesh(core_axis_name="c", subcore_axis_name="s")

    @functools.partial(
        pl.kernel,
        out_type=(
            jax.ShapeDtypeStruct((4, N, CHUNK), jnp.float32),
            jax.ShapeDtypeStruct((NHP, CHUNK), jnp.float32),
            jax.ShapeDtypeStruct((NHP, CHUNK), jnp.float32),
            jax.ShapeDtypeStruct((NHP, CHUNK), jnp.float32),
            jax.ShapeDtypeStruct((NHP, CHUNK), jnp.float32),
        ),
        mesh=mesh,
        compiler_params=pltpu.CompilerParams(
            needs_layout_passes=False, use_tc_tiling_on_sc=False),
        scratch_types=dict(
            deg_sh=pltpu.VMEM_SHARED((NPAD,), jnp.float32),
            out_sh=pltpu.VMEM_SHARED((NACC, CHUNK), jnp.float32),
            disb=pltpu.VMEM((DSLICE,), jnp.float32),
            dchunk=pltpu.VMEM((ROW_CHUNK + 16,), jnp.float32),
            rowp=pltpu.VMEM((2, RB, B), jnp.int32),
            colp=pltpu.VMEM((2, RB, B), jnp.int32),
            rowb=pltpu.VMEM((2, B), jnp.int32),
            colb=pltpu.VMEM((2, B), jnp.int32),
            oneb=pltpu.VMEM((B,), jnp.float32),
            gbuf=pltpu.VMEM((2, B, CHUNK), jnp.float32),
            gsem=pltpu.SemaphoreType.DMA,
            dsem=pltpu.SemaphoreType.DMA,
            isem=pltpu.SemaphoreType.DMA,
            ssem=pltpu.SemaphoreType.DMA,
        ),
    )
    def k(row_hbm, col_hbm, h0_hbm, h1_hbm, h2_hbm,
          out_hbm, hp0_hbm, hp1a_hbm, hp1b_hbm, hp2_hbm,
          deg_sh, out_sh, disb, dchunk, rowp, colp, rowb, colb, oneb, gbuf,
          gsem, dsem, isem, ssem):
        c = lax.axis_index("c")
        s = lax.axis_index("s")

        def issue_round(b_start, r, pb):
            pltpu.async_copy(row_hbm.at[pl.ds(b_start + r * RB, RB)],
                             rowp.at[pb], isem)
            pltpu.async_copy(col_hbm.at[pl.ds(b_start + r * RB, RB)],
                             colp.at[pb], isem)

        def finish_round(b_start, r, pb):
            pltpu.make_async_copy(row_hbm.at[pl.ds(b_start + r * RB, RB)],
                                  rowp.at[pb], isem).wait()
            pltpu.make_async_copy(col_hbm.at[pl.ds(b_start + r * RB, RB)],
                                  colp.at[pb], isem).wait()

            @pl.loop(0, RB)
            def _(jb):
                for v in range(B // 16):
                    sl = pl.ds(16 * v, 16)
                    rv = rowp[pb, jb, sl]
                    cv = colp[pb, jb, sl]
                    nonself = rv != cv
                    rowp[pb, jb, sl] = jnp.where(nonself, rv, N)
                    colp[pb, jb, sl] = jnp.where(nonself, cv, N)

        def load_remainder(batch_id):
            pltpu.sync_copy(row_hbm.at[pl.ds(batch_id, 1)],
                            rowb.at[pl.ds(0, 1)])
            pltpu.sync_copy(col_hbm.at[pl.ds(batch_id, 1)],
                            colb.at[pl.ds(0, 1)])
            for v in range(B // 16):
                sl = pl.ds(16 * v, 16)
                rv = rowb[0, sl]
                cv = colb[0, sl]
                nonself = rv != cv
                rowb[0, sl] = jnp.where(nonself, rv, N)
                colb[0, sl] = jnp.where(nonself, cv, N)

        @pl.loop(0, DSLICE // 16)
        def _(j):
            disb[pl.ds(16 * j, 16)] = jnp.zeros((16,), jnp.float32)

        pltpu.sync_copy(disb, deg_sh.at[pl.ds(s * DSLICE, DSLICE)])
        for v in range(B // 16):
            oneb[pl.ds(16 * v, 16)] = jnp.full((16,), 1.0, jnp.float32)

        plsc.subcore_barrier()

        issue_round(NBQ * s, 0, 0)

        @pl.loop(0, NBQ // RB)
        def _(r):
            pb = r % 2
            finish_round(NBQ * s, r, pb)

            @pl.when(r + 1 < NBQ // RB)
            def _():
                issue_round(NBQ * s, r + 1, 1 - pb)

            @pl.loop(0, RB)
            def _(jb):
                pltpu.async_copy(oneb, deg_sh.at[rowp.at[pb, jb]], dsem,
                                 add=True)

            @pl.loop(0, RB)
            def _(jb):
                pltpu.make_async_copy(oneb, deg_sh.at[rowp.at[pb, jb]],
                                      dsem).wait()

        @pl.when(s < NB_REM)
        def _():
            load_remainder(NB - NB_REM + s)
            pltpu.sync_copy(oneb, deg_sh.at[rowb.at[0]], add=True)

        plsc.subcore_barrier()

        pltpu.sync_copy(deg_sh.at[pl.ds(s * DSLICE, DSLICE)], disb)

        @pl.loop(0, DSLICE // 16)
        def _(j):
            sl = pl.ds(16 * j, 16)
            disb[sl] = _rsqrt16(disb[sl] + 1.0)

        pltpu.sync_copy(disb, deg_sh.at[pl.ds(s * DSLICE, DSLICE)])
        plsc.subcore_barrier()

        nch_s = (NCHUNK_FULL // NSUB) + jnp.where(s < NCHUNK_FULL % NSUB, 1, 0)

        def scaled_rows(h_src, base, nrows, dst):
            pltpu.sync_copy(h_src.at[pl.ds(base, nrows)],
                            dst.at[0].at[pl.ds(0, nrows)])
            pltpu.sync_copy(deg_sh.at[pl.ds(base, ROW_CHUNK + 16)], dchunk)

            @pl.loop(0, nrows)
            def _(r):
                d = dchunk[pl.ds(r, 16)][0]
                for v in range(VECS):
                    sl = pl.ds(16 * v, 16)
                    dst[0, r, sl] = dst[0, r, sl] * d

        def for_own_rows(fn):
            @pl.loop(0, nch_s)
            def _(t):
                base = pl.multiple_of(ROW_CHUNK * (s + NSUB * t), ROW_CHUNK)
                fn(base, ROW_CHUNK)

            @pl.when(s == NSUB - 1)
            def _():
                fn(TAIL_BASE, TAIL_ROWS)

        def edge_pass(h_hbm, hp_hbm, out_idx, b_base, nbq, n_rem, self_init):
            def scale_chunk(base, nrows):
                scaled_rows(h_hbm, base, nrows, gbuf)
                pltpu.sync_copy(gbuf.at[0].at[pl.ds(0, nrows)],
                                hp_hbm.at[pl.ds(base, nrows)])
                if self_init:
                    pltpu.sync_copy(gbuf.at[0].at[pl.ds(0, nrows)],
                                    out_sh.at[pl.ds(base, nrows)])

            for_own_rows(scale_chunk)

            if not self_init:
                @pl.loop(0, B)
                def _(r):
                    for v in range(VECS):
                        gbuf[1, r, pl.ds(16 * v, 16)] = (
                            jnp.zeros((16,), jnp.float32))

                def zero_chunk(base, nrows):
                    pltpu.sync_copy(gbuf.at[1].at[pl.ds(0, nrows)],
                                    out_sh.at[pl.ds(base, nrows)])

                for_own_rows(zero_chunk)

            plsc.subcore_barrier()

            issue_round(b_base + nbq * s, 0, 0)

            @pl.loop(0, nbq // RB)
            def _(r):
                pb = r % 2
                finish_round(b_base + nbq * s, r, pb)

                @pl.when(r + 1 < nbq // RB)
                def _():
                    issue_round(b_base + nbq * s, r + 1, 1 - pb)

                pltpu.async_copy(hp_hbm.at[rowp.at[pb, 0]], gbuf.at[0], gsem)

                @pl.loop(0, RB, step=2)
                def _(jb):
                    for bslot in range(2):
                        j = jb + bslot
                        nxt = j + 1

                        @pl.when(r * RB + j >= 1)
                        def _():
                            pltpu.make_async_copy(
                                gbuf.at[1 - bslot],
                                out_sh.at[colp.at[pb, 0]], ssem).wait()

                        @pl.when(nxt < RB)
                        def _():
                            pltpu.async_copy(hp_hbm.at[rowp.at[pb, nxt]],
                                             gbuf.at[1 - bslot], gsem)

                        pltpu.make_async_copy(hp_hbm.at[rowp.at[pb, j]],
                                              gbuf.at[bslot], gsem).wait()
                        pltpu.async_copy(gbuf.at[bslot],
                                         out_sh.at[colp.at[pb, j]], ssem,
                                         add=True)

            pltpu.make_async_copy(gbuf.at[0], out_sh.at[colp.at[0, 0]],
                                  ssem).wait()

            @pl.when(s < n_rem)
            def _():
                load_remainder(b_base + NSUB * nbq + s)
                pltpu.sync_copy(hp_hbm.at[rowb.at[0]], gbuf.at[0])
                pltpu.sync_copy(gbuf.at[0], out_sh.at[colb.at[0]], add=True)

            plsc.subcore_barrier()

            def writeout_chunk(base, nrows):
                scaled_rows(out_sh, base, nrows, gbuf)
                pltpu.sync_copy(gbuf.at[0].at[pl.ds(0, nrows)],
                                out_hbm.at[out_idx].at[pl.ds(base, nrows)])

            for_own_rows(writeout_chunk)

        @pl.when(c == 0)
        def _():
            edge_pass(h0_hbm, hp0_hbm, 0, 0, NBQ, NB_REM, True)
            plsc.subcore_barrier()
            edge_pass(h1_hbm, hp1a_hbm, 1, 0, NBQH, NBH_REM, True)

        @pl.when(c == 1)
        def _():
            edge_pass(h2_hbm, hp2_hbm, 3, 0, NBQ, NB_REM, True)
            plsc.subcore_barrier()
            edge_pass(h1_hbm, hp1b_hbm, 2, NBH, NBQH, NBH_REM, False)

    return k(row2d, col2d, h0, h1, h2)


def kernel(x, edge_index, W, b):
    w_pad = jnp.pad(W, ((0, 0), (0, OUT_PAD - OUT)))
    b_pad = jnp.pad(b, (0, OUT_PAD - OUT)).reshape(1, OUT_PAD)
    h0, h1, h2 = _tc_linear(x, w_pad, b_pad)
    row2d = edge_index[0].astype(jnp.int32).reshape(NB, B)
    col2d = edge_index[1].astype(jnp.int32).reshape(NB, B)
    out4, _, _, _, _ = _sc_gcn(row2d, col2d, h0, h1, h2)
    return _tc_pack(out4)
